# Initial kernel scaffold; baseline (speedup 1.0000x reference)
import jax
import jax.numpy as jnp
from jax.experimental import pallas as pl


def kernel(X, edge_index, W1, b1, W2, b2):
    raise NotImplementedError("write your pallas kernel here")


# speedup vs baseline: 5.3308x; 5.3308x over previous
import functools

import jax
import jax.numpy as jnp
from jax import lax
from jax.experimental import pallas as pl
from jax.experimental.pallas import tpu as pltpu
from jax.experimental.pallas import tpu_sc as plsc

N = 10000
E = 320000
D = 128
DH = D // 2

NC = 2
NS = 16
EPT = E // NS
CHUNK = 80
NCHUNK = EPT // CHUNK
ROWS_PER_TILE = 640
N_PAD = NS * ROWS_PER_TILE


def _make_sc_aggregate(with_deg):
  agg_ty = jax.ShapeDtypeStruct((NC, N_PAD, DH), jnp.float32)
  if with_deg:
    out_type = [agg_ty, jax.ShapeDtypeStruct((N_PAD, 16), jnp.float32)]
  else:
    out_type = agg_ty

  scratch_types = [
      pltpu.VMEM((NCHUNK, CHUNK), jnp.int32),
      pltpu.VMEM((NCHUNK, CHUNK), jnp.int32),
      pltpu.VMEM((CHUNK, DH), jnp.float32),
      pltpu.VMEM((128, DH), jnp.float32),
      pltpu.VMEM_SHARED((N_PAD, DH), jnp.float32),
      pltpu.SemaphoreType.DMA,
  ]
  if with_deg:
    scratch_types += [
        pltpu.VMEM((CHUNK, 16), jnp.float32),
        pltpu.VMEM((128, 16), jnp.float32),
        pltpu.VMEM_SHARED((N_PAD, 16), jnp.float32),
    ]

  mesh = plsc.VectorSubcoreMesh(core_axis_name="c", subcore_axis_name="s")

  @functools.partial(
      pl.kernel, mesh=mesh, out_type=out_type, scratch_types=scratch_types,
      compiler_params=pltpu.CompilerParams(use_tc_tiling_on_sc=False))
  def sc_agg(x_hbm, src_hbm, dst_hbm, *rest):
    if with_deg:
      (agg_out, deg_out, src_v, dst_v, rows_v, zeros_v, agg_sp, sem,
       ones_v, zeros16_v, deg_sp) = rest
    else:
      (agg_out, src_v, dst_v, rows_v, zeros_v, agg_sp, sem) = rest
    cid = lax.axis_index("c")
    sid = lax.axis_index("s")

    def zfill(i, carry):
      z = jnp.zeros((16,), jnp.float32)
      for j in range(DH // 16):
        zeros_v[i, pl.ds(j * 16, 16)] = z
      if with_deg:
        zeros16_v[i, :] = z
      return carry
    lax.fori_loop(0, 128, zfill, 0)
    if with_deg:
      def ofill(i, carry):
        ones_v[i, :] = jnp.full((16,), 1.0, jnp.float32)
        return carry
      lax.fori_loop(0, CHUNK, ofill, 0)

    for k in range(ROWS_PER_TILE // 128):
      rows = pl.ds(sid * ROWS_PER_TILE + k * 128, 128)
      pltpu.sync_copy(zeros_v, agg_sp.at[rows])
      if with_deg:
        @pl.when(cid == 0)
        def _():
          pltpu.sync_copy(zeros16_v, deg_sp.at[rows])
    plsc.subcore_barrier()

    pltpu.sync_copy(src_hbm.at[cid, sid], src_v)
    pltpu.sync_copy(dst_hbm.at[sid], dst_v)

    def chunk_body(c, carry):
      src_row = src_v.at[c]
      dst_row = dst_v.at[c]
      pltpu.async_copy(x_hbm.at[src_row], rows_v, sem).wait()
      pltpu.sync_copy(rows_v, agg_sp.at[dst_row], add=True)
      if with_deg:
        @pl.when(cid == 0)
        def _():
          pltpu.sync_copy(ones_v, deg_sp.at[dst_row], add=True)
      return carry
    lax.fori_loop(0, NCHUNK, chunk_body, 0)

    plsc.subcore_barrier()

    for k in range(ROWS_PER_TILE // 128):
      rows = pl.ds(sid * ROWS_PER_TILE + k * 128, 128)
      pltpu.sync_copy(agg_sp.at[rows], zeros_v)
      pltpu.sync_copy(zeros_v, agg_out.at[cid, rows])
      if with_deg:
        @pl.when(cid == 0)
        def _():
          pltpu.sync_copy(deg_sp.at[rows], zeros16_v)
          pltpu.sync_copy(zeros16_v, deg_out.at[rows])

  return sc_agg


_sc_agg_deg = _make_sc_aggregate(with_deg=True)
_sc_agg = _make_sc_aggregate(with_deg=False)


def _tc_layer(xs, agg, deg, w, b, relu, split_out):
  R = 400
  grid = (N // R,)

  def body(x0_ref, x1_ref, a0_ref, a1_ref, d_ref, wt_ref, wb_ref, b_ref,
           o_ref):
    xv = jnp.concatenate([x0_ref[0], x1_ref[0]], axis=1)
    a = jnp.concatenate([a0_ref[0], a1_ref[0]], axis=1)
    d = d_ref[:, 0:1]
    xn = a / jnp.maximum(d, 1.0)
    h = (jnp.dot(xv, wt_ref[0], preferred_element_type=jnp.float32)
         + jnp.dot(xn, wb_ref[0], preferred_element_type=jnp.float32)
         + b_ref[...])
    if relu:
      h = jnp.maximum(h, 0.0)
    if split_out:
      o_ref[0] = h[:, :DH]
      o_ref[1] = h[:, DH:]
    else:
      o_ref[...] = h

  if split_out:
    out_shape = jax.ShapeDtypeStruct((NC, N, DH), jnp.float32)
    out_specs = pl.BlockSpec((NC, R, DH), lambda i: (0, i, 0))
  else:
    out_shape = jax.ShapeDtypeStruct((N, D), jnp.float32)
    out_specs = pl.BlockSpec((R, D), lambda i: (i, 0))

  return pl.pallas_call(
      body,
      grid=grid,
      in_specs=[
          pl.BlockSpec((1, R, DH), lambda i: (0, i, 0)),
          pl.BlockSpec((1, R, DH), lambda i: (1, i, 0)),
          pl.BlockSpec((1, R, DH), lambda i: (0, i, 0)),
          pl.BlockSpec((1, R, DH), lambda i: (1, i, 0)),
          pl.BlockSpec((R, 16), lambda i: (i, 0)),
          pl.BlockSpec((1, D, D), lambda i: (0, 0, 0)),
          pl.BlockSpec((1, D, D), lambda i: (1, 0, 0)),
          pl.BlockSpec((1, D), lambda i: (0, 0)),
      ],
      out_specs=out_specs,
      out_shape=out_shape,
  )(xs, xs, agg, agg, deg, w, w, b)


@jax.jit
def kernel(X, edge_index, W1, b1, W2, b2):
  src = edge_index[0]
  dst3 = edge_index[1].reshape(NS, NCHUNK, CHUNK)
  src_off = jnp.stack([src, src + N]).reshape(NC, NS, NCHUNK, CHUNK)
  xs = X.reshape(N, NC, DH).transpose(1, 0, 2)
  w1 = W1.reshape(2, D, D)
  w2 = W2.reshape(2, D, D)

  agg1, deg = _sc_agg_deg(xs.reshape(NC * N, DH), src_off, dst3)
  h1s = _tc_layer(xs, agg1, deg, w1, b1.reshape(1, D), relu=True,
                  split_out=True)
  agg2 = _sc_agg(h1s.reshape(NC * N, DH), src_off, dst3)
  H2 = _tc_layer(h1s, agg2, deg, w2, b2.reshape(1, D), relu=False,
                 split_out=False)
  return H2
